# Initial kernel scaffold; baseline (speedup 1.0000x reference)
import jax
import jax.numpy as jnp
from jax.experimental import pallas as pl


def kernel(x, edge_index, rel, norm, weight, loop_weight):
    raise NotImplementedError("write your pallas kernel here")


# speedup vs baseline: 14.3843x; 14.3843x over previous
import functools

import jax
import jax.numpy as jnp
from jax import lax
from jax.experimental import pallas as pl
from jax.experimental.pallas import tpu as pltpu
from jax.experimental.pallas import tpu_sc as plsc

NC = 2
NS = 16
NW = NC * NS
L = 16

T = 256
CH = 128
CS = 80


def _plan(rel, src, dst, R, E, Ep):
    rel_s, src_s, dst_s = lax.sort((rel, src, dst), num_keys=1)
    bounds = jnp.searchsorted(
        rel_s, jnp.arange(R + 1, dtype=jnp.int32), side="left"
    ).astype(jnp.int32)
    off = bounds[:-1]
    counts = bounds[1:] - off
    nt = (counts + (T - 1)) // T
    tile_cum = jnp.cumsum(nt).astype(jnp.int32)
    padded_off = (tile_cum - nt) * T
    delta = padded_off - off
    NT = Ep // T
    tail_delta = jnp.full((1,), Ep - CH - E, dtype=jnp.int32)
    delta_tab = jnp.concatenate([delta, tail_delta, jnp.zeros((7,), jnp.int32)])
    tile_rel = jnp.minimum(
        jnp.searchsorted(tile_cum, jnp.arange(NT, dtype=jnp.int32), side="right"),
        R - 1,
    ).astype(jnp.int32)
    src_p = jnp.concatenate([src_s, jnp.zeros((CH,), jnp.int32)])
    rel_p = jnp.concatenate([rel_s, jnp.full((CH,), R, jnp.int32)])
    return src_p, rel_p, rel_s, dst_s, delta_tab, tile_rel


def _plan_dst(dst_s, rel_s, N, E, NPW):
    iota = jnp.arange(E, dtype=jnp.int32)
    dst_d, j_d, rel_d = lax.sort((dst_s, iota, rel_s), num_keys=1)
    q = jnp.searchsorted(
        dst_d, jnp.arange(NW + 1, dtype=jnp.int32) * NPW, side="left"
    ).astype(jnp.int32)
    q8 = (q[:NW] // 16) * 16
    nchv = (q[1:] - q8 + CS - 1) // CS
    QN = 48
    zpad = jnp.zeros((QN - NW,), jnp.int32)
    q8 = jnp.concatenate([q8, zpad])
    nchv = jnp.concatenate([nchv, zpad])
    dstp = jnp.concatenate([dst_d, jnp.full((CS,), N, jnp.int32)])
    jp = jnp.concatenate([j_d, jnp.zeros((CS,), jnp.int32)])
    relp = jnp.concatenate([rel_d, jnp.zeros((CS,), jnp.int32)])
    return dstp, jp, relp, q8, nchv


def _sc_gather(x, src_p, rel_p, delta_tab, E, Ep, D):
    per_w = -(-(-(-E // NW)) // CH) * CH
    nch_full = per_w // CH
    RT = delta_tab.shape[0]
    mesh = plsc.VectorSubcoreMesh(core_axis_name="c", subcore_axis_name="s")

    @functools.partial(
        pl.kernel,
        out_type=jax.ShapeDtypeStruct((Ep, D), jnp.float32),
        mesh=mesh,
        scratch_types=[
            pltpu.VMEM((CH,), jnp.int32),
            pltpu.VMEM((CH,), jnp.int32),
            pltpu.VMEM((1, CH), jnp.int32),
            pltpu.VMEM((RT,), jnp.int32),
            pltpu.VMEM((CH, D), jnp.float32),
            pltpu.SemaphoreType.DMA,
        ],
        compiler_params=pltpu.CompilerParams(needs_layout_passes=False),
    )
    def k(x_hbm, src_hbm, rel_hbm, dtab_hbm, xs_hbm,
          src_v, rel_v, slot_v, dtab_v, rows_v, sem):
        wid = lax.axis_index("s") * NC + lax.axis_index("c")
        base = wid * per_w
        rem = jnp.maximum(E - base, 0)
        nch = jnp.minimum(nch_full, (rem + CH - 1) // CH)
        pltpu.sync_copy(dtab_hbm, dtab_v)

        def chunk(c, _):
            start = pl.multiple_of(base + c * CH, CH)
            pltpu.sync_copy(src_hbm.at[pl.ds(start, CH)], src_v)
            pltpu.sync_copy(rel_hbm.at[pl.ds(start, CH)], rel_v)
            for kk in range(CH // L):
                r16 = rel_v[pl.ds(kk * L, L)]
                d16 = plsc.load_gather(dtab_v, [r16])
                slot_v[0, pl.ds(kk * L, L)] = (
                    d16 + start + kk * L + lax.iota(jnp.int32, L)
                )
            pltpu.async_copy(x_hbm.at[src_v], rows_v, sem).wait()
            pltpu.async_copy(rows_v, xs_hbm.at[slot_v.at[0]], sem).wait()
            return _

        lax.fori_loop(0, nch, chunk, None)

    return k(x, src_p, rel_p, delta_tab)


def _tc_bmm(xs, w4, tile_rel, Ep, D, NB, SI, SO):
    NT = Ep // T

    def body(tr_ref, xs_ref, w_ref, out_ref):
        for b in range(NB):
            out_ref[:, b * SO:(b + 1) * SO] = jnp.dot(
                xs_ref[:, b * SI:(b + 1) * SI],
                w_ref[0, b],
                preferred_element_type=jnp.float32,
            )

    grid_spec = pltpu.PrefetchScalarGridSpec(
        num_scalar_prefetch=1,
        grid=(NT,),
        in_specs=[
            pl.BlockSpec((T, D), lambda t, tr: (t, 0)),
            pl.BlockSpec((1, NB, SI, SO), lambda t, tr: (tr[t], 0, 0, 0)),
        ],
        out_specs=pl.BlockSpec((T, D), lambda t, tr: (t, 0)),
    )
    return pl.pallas_call(
        body,
        grid_spec=grid_spec,
        out_shape=jax.ShapeDtypeStruct((Ep, D), jnp.float32),
    )(tile_rel, xs, w4)


def _scalar_at(vref, i, ngroups):
    total = jnp.zeros((), jnp.int32)
    for g in range(ngroups):
        v = vref[pl.ds(g * L, L)]
        lane = lax.iota(jnp.int32, L) + g * L
        total = total + jnp.sum(jnp.where(lane == i, v, 0))
    return total


def _sc_scatter(msg, dstp, jp, relp, q8, nchv, delta_tab, zrows, N, E, D, NPW):
    RT = delta_tab.shape[0]
    QN = q8.shape[0]
    ACC = NPW + 64
    zw = zrows.shape[0]
    last_rows = N - (NW - 1) * NPW
    mesh = plsc.VectorSubcoreMesh(core_axis_name="c", subcore_axis_name="s")

    @functools.partial(
        pl.kernel,
        out_type=jax.ShapeDtypeStruct((N, D), jnp.float32),
        mesh=mesh,
        scratch_types=[
            pltpu.VMEM((CS,), jnp.int32),
            pltpu.VMEM((CS,), jnp.int32),
            pltpu.VMEM((CS,), jnp.int32),
            pltpu.VMEM((CS,), jnp.int32),
            pltpu.VMEM((CS,), jnp.int32),
            pltpu.VMEM((RT,), jnp.int32),
            pltpu.VMEM((QN,), jnp.int32),
            pltpu.VMEM((QN,), jnp.int32),
            pltpu.VMEM((CS, D), jnp.float32),
            pltpu.VMEM((ACC, D), jnp.float32),
            pltpu.SemaphoreType.DMA,
        ],
        compiler_params=pltpu.CompilerParams(needs_layout_passes=False),
    )
    def k(msg_hbm, dst_hbm, j_hbm, rel_hbm, q8_hbm, nch_hbm, dtab_hbm, z_hbm,
          out_hbm, dst_v, j_v, rel_v, slot_v, lidx_v, dtab_v, q_v, n_v,
          rows_v, acc, sem):
        wid = lax.axis_index("s") * NC + lax.axis_index("c")
        base_node = wid * NPW
        pltpu.sync_copy(dtab_hbm, dtab_v)
        pltpu.sync_copy(q8_hbm, q_v)
        pltpu.sync_copy(nch_hbm, n_v)
        s0 = _scalar_at(q_v, wid, QN // L)
        nch = _scalar_at(n_v, wid, QN // L)
        for i in range(ACC // zw):
            pltpu.sync_copy(z_hbm, acc.at[pl.ds(i * zw, zw)])

        def chunk(c, _):
            start = pl.multiple_of(s0 + c * CS, 8)
            pltpu.sync_copy(dst_hbm.at[pl.ds(start, CS)], dst_v)
            pltpu.sync_copy(j_hbm.at[pl.ds(start, CS)], j_v)
            pltpu.sync_copy(rel_hbm.at[pl.ds(start, CS)], rel_v)
            for kk in range(CS // L):
                sl = pl.ds(kk * L, L)
                r16 = rel_v[sl]
                slot_v[sl] = j_v[sl] + plsc.load_gather(dtab_v, [r16])
                d16 = dst_v[sl]
                dloc = d16 - base_node
                inr = (dloc >= 0) & (dloc < NPW)
                lidx_v[sl] = jnp.where(inr, dloc, NPW + (d16 & 63))
            pltpu.async_copy(msg_hbm.at[slot_v], rows_v, sem).wait()

            def grp(g, _2):
                d16 = lidx_v[pl.ds(g * L, L)]
                for j in range(L):
                    row16 = jnp.zeros((L,), jnp.int32) + d16[j]
                    erow16 = jnp.zeros((L,), jnp.int32) + (g * L + j)
                    for cc in range(D // L):
                        col16 = cc * L + lax.iota(jnp.int32, L)
                        vals = plsc.load_gather(rows_v, [erow16, col16])
                        plsc.addupdate_scatter(acc, [row16, col16], vals)
                return _2

            lax.fori_loop(0, CS // L, grp, None)
            return _

        lax.fori_loop(0, nch, chunk, None)

        @pl.when(wid < NW - 1)
        def _full():
            pltpu.sync_copy(
                acc.at[pl.ds(0, NPW)],
                out_hbm.at[pl.ds(base_node, NPW)],
            )

        @pl.when(wid == NW - 1)
        def _tail():
            pltpu.sync_copy(
                acc.at[pl.ds(0, last_rows)],
                out_hbm.at[pl.ds((NW - 1) * NPW, last_rows)],
            )

    return k(msg, dstp, jp, relp, q8, nchv, delta_tab, zrows)


def _tc_finish(acc, norm, x, loop_weight, N, D):
    BR = 400
    grid = (N // BR,)

    def body(acc_ref, norm_ref, x_ref, lw_ref, out_ref):
        out_ref[...] = acc_ref[...] * norm_ref[...] + jnp.dot(
            x_ref[...], lw_ref[...], preferred_element_type=jnp.float32
        )

    return pl.pallas_call(
        body,
        grid=grid,
        in_specs=[
            pl.BlockSpec((BR, D), lambda i: (i, 0)),
            pl.BlockSpec((BR, 1), lambda i: (i, 0)),
            pl.BlockSpec((BR, D), lambda i: (i, 0)),
            pl.BlockSpec((D, D), lambda i: (0, 0)),
        ],
        out_specs=pl.BlockSpec((BR, D), lambda i: (i, 0)),
        out_shape=jax.ShapeDtypeStruct((N, D), jnp.float32),
    )(acc, norm, x, loop_weight)


def kernel(x, edge_index, rel, norm, weight, loop_weight):
    N, D = x.shape
    E = edge_index.shape[1]
    R = weight.shape[0]
    NB = 4
    SI = SO = D // NB
    NT = E // T + R + 1
    Ep = NT * T

    NPW = (-(-N // NW) + 7) // 8 * 8

    src = edge_index[0]
    dst = edge_index[1]
    src_p, rel_p, rel_s, dst_s, delta_tab, tile_rel = _plan(
        rel, src, dst, R, E, Ep)
    dstp, jp, relp, q8, nchv = _plan_dst(dst_s, rel_s, N, E, NPW)
    w4 = weight.reshape(R, NB, SI, SO)
    zrows = jnp.zeros((64, D), jnp.float32)

    xs = _sc_gather(x, src_p, rel_p, delta_tab, E, Ep, D)
    msg = _tc_bmm(xs, w4, tile_rel, Ep, D, NB, SI, SO)
    acc = _sc_scatter(msg, dstp, jp, relp, q8, nchv, delta_tab, zrows,
                      N, E, D, NPW)
    return _tc_finish(acc, norm, x, loop_weight, N, D)


# speedup vs baseline: 15.1800x; 1.0553x over previous
import functools

import jax
import jax.numpy as jnp
from jax import lax
from jax.experimental import pallas as pl
from jax.experimental.pallas import tpu as pltpu
from jax.experimental.pallas import tpu_sc as plsc

NC = 2
NS = 16
NW = NC * NS
L = 16

T = 256
CH = 128
CS = 80


def _plan(rel, src, dst, R, E, Ep):
    key, dst_s = lax.sort(((rel << 14) | src, dst), num_keys=1)
    rel_s = key >> 14
    src_s = key & 0x3FFF
    bounds = jnp.searchsorted(
        rel_s, jnp.arange(R + 1, dtype=jnp.int32), side="left"
    ).astype(jnp.int32)
    off = bounds[:-1]
    counts = bounds[1:] - off
    nt = (counts + (T - 1)) // T
    tile_cum = jnp.cumsum(nt).astype(jnp.int32)
    padded_off = (tile_cum - nt) * T
    delta = padded_off - off
    NT = Ep // T
    tail_delta = jnp.full((1,), Ep - CH - E, dtype=jnp.int32)
    delta_tab = jnp.concatenate([delta, tail_delta, jnp.zeros((7,), jnp.int32)])
    tile_rel = jnp.minimum(
        jnp.searchsorted(tile_cum, jnp.arange(NT, dtype=jnp.int32), side="right"),
        R - 1,
    ).astype(jnp.int32)
    src_p = jnp.concatenate([src_s, jnp.zeros((CH,), jnp.int32)])
    rel_p = jnp.concatenate([rel_s, jnp.full((CH,), R, jnp.int32)])
    return src_p, rel_p, rel_s, dst_s, delta_tab, tile_rel


def _plan_dst(dst_s, rel_s, N, E, NPW):
    iota = jnp.arange(E, dtype=jnp.int32)
    key2, rel_d = lax.sort(
        ((dst_s.astype(jnp.uint32) << 18) | iota.astype(jnp.uint32), rel_s),
        num_keys=1)
    dst_d = (key2 >> 18).astype(jnp.int32)
    j_d = (key2 & 0x3FFFF).astype(jnp.int32)
    q = jnp.searchsorted(
        dst_d, jnp.arange(NW + 1, dtype=jnp.int32) * NPW, side="left"
    ).astype(jnp.int32)
    q8 = (q[:NW] // 16) * 16
    nchv = (q[1:] - q8 + CS - 1) // CS
    QN = 48
    zpad = jnp.zeros((QN - NW,), jnp.int32)
    q8 = jnp.concatenate([q8, zpad])
    nchv = jnp.concatenate([nchv, zpad])
    dstp = jnp.concatenate([dst_d, jnp.full((CS,), N, jnp.int32)])
    jp = jnp.concatenate([j_d, jnp.zeros((CS,), jnp.int32)])
    relp = jnp.concatenate([rel_d, jnp.zeros((CS,), jnp.int32)])
    return dstp, jp, relp, q8, nchv


def _sc_gather(x, src_p, rel_p, delta_tab, E, Ep, D):
    per_w = -(-(-(-E // NW)) // CH) * CH
    nch_full = per_w // CH
    RT = delta_tab.shape[0]
    mesh = plsc.VectorSubcoreMesh(core_axis_name="c", subcore_axis_name="s")

    @functools.partial(
        pl.kernel,
        out_type=jax.ShapeDtypeStruct((Ep, D), jnp.float32),
        mesh=mesh,
        scratch_types=[
            pltpu.VMEM((CH,), jnp.int32),
            pltpu.VMEM((CH,), jnp.int32),
            pltpu.VMEM((1, CH), jnp.int32),
            pltpu.VMEM((RT,), jnp.int32),
            pltpu.VMEM((CH, D), jnp.float32),
            pltpu.SemaphoreType.DMA,
        ],
        compiler_params=pltpu.CompilerParams(needs_layout_passes=False),
    )
    def k(x_hbm, src_hbm, rel_hbm, dtab_hbm, xs_hbm,
          src_v, rel_v, slot_v, dtab_v, rows_v, sem):
        wid = lax.axis_index("s") * NC + lax.axis_index("c")
        base = wid * per_w
        rem = jnp.maximum(E - base, 0)
        nch = jnp.minimum(nch_full, (rem + CH - 1) // CH)
        pltpu.sync_copy(dtab_hbm, dtab_v)

        def chunk(c, _):
            start = pl.multiple_of(base + c * CH, CH)
            pltpu.sync_copy(src_hbm.at[pl.ds(start, CH)], src_v)
            pltpu.sync_copy(rel_hbm.at[pl.ds(start, CH)], rel_v)
            for kk in range(CH // L):
                r16 = rel_v[pl.ds(kk * L, L)]
                d16 = plsc.load_gather(dtab_v, [r16])
                slot_v[0, pl.ds(kk * L, L)] = (
                    d16 + start + kk * L + lax.iota(jnp.int32, L)
                )
            pltpu.async_copy(x_hbm.at[src_v], rows_v, sem).wait()
            pltpu.async_copy(rows_v, xs_hbm.at[slot_v.at[0]], sem).wait()
            return _

        lax.fori_loop(0, nch, chunk, None)

    return k(x, src_p, rel_p, delta_tab)


def _tc_bmm(xs, w4, tile_rel, Ep, D, NB, SI, SO):
    NT = Ep // T

    def body(tr_ref, xs_ref, w_ref, out_ref):
        for b in range(NB):
            out_ref[:, b * SO:(b + 1) * SO] = jnp.dot(
                xs_ref[:, b * SI:(b + 1) * SI],
                w_ref[0, b],
                preferred_element_type=jnp.float32,
            )

    grid_spec = pltpu.PrefetchScalarGridSpec(
        num_scalar_prefetch=1,
        grid=(NT,),
        in_specs=[
            pl.BlockSpec((T, D), lambda t, tr: (t, 0)),
            pl.BlockSpec((1, NB, SI, SO), lambda t, tr: (tr[t], 0, 0, 0)),
        ],
        out_specs=pl.BlockSpec((T, D), lambda t, tr: (t, 0)),
    )
    return pl.pallas_call(
        body,
        grid_spec=grid_spec,
        out_shape=jax.ShapeDtypeStruct((Ep, D), jnp.float32),
    )(tile_rel, xs, w4)


def _scalar_at(vref, i, ngroups):
    total = jnp.zeros((), jnp.int32)
    for g in range(ngroups):
        v = vref[pl.ds(g * L, L)]
        lane = lax.iota(jnp.int32, L) + g * L
        total = total + jnp.sum(jnp.where(lane == i, v, 0))
    return total


def _sc_scatter(msg, dstp, jp, relp, q8, nchv, delta_tab, zrows, N, E, D, NPW):
    RT = delta_tab.shape[0]
    QN = q8.shape[0]
    ACC = NPW + 64
    zw = zrows.shape[0]
    last_rows = N - (NW - 1) * NPW
    mesh = plsc.VectorSubcoreMesh(core_axis_name="c", subcore_axis_name="s")

    @functools.partial(
        pl.kernel,
        out_type=jax.ShapeDtypeStruct((N, D), jnp.float32),
        mesh=mesh,
        scratch_types=[
            pltpu.VMEM((CS,), jnp.int32),
            pltpu.VMEM((CS,), jnp.int32),
            pltpu.VMEM((CS,), jnp.int32),
            pltpu.VMEM((CS,), jnp.int32),
            pltpu.VMEM((CS,), jnp.int32),
            pltpu.VMEM((RT,), jnp.int32),
            pltpu.VMEM((QN,), jnp.int32),
            pltpu.VMEM((QN,), jnp.int32),
            pltpu.VMEM((CS, D), jnp.float32),
            pltpu.VMEM((ACC, D), jnp.float32),
            pltpu.SemaphoreType.DMA,
        ],
        compiler_params=pltpu.CompilerParams(needs_layout_passes=False),
    )
    def k(msg_hbm, dst_hbm, j_hbm, rel_hbm, q8_hbm, nch_hbm, dtab_hbm, z_hbm,
          out_hbm, dst_v, j_v, rel_v, slot_v, lidx_v, dtab_v, q_v, n_v,
          rows_v, acc, sem):
        wid = lax.axis_index("s") * NC + lax.axis_index("c")
        base_node = wid * NPW
        pltpu.sync_copy(dtab_hbm, dtab_v)
        pltpu.sync_copy(q8_hbm, q_v)
        pltpu.sync_copy(nch_hbm, n_v)
        s0 = _scalar_at(q_v, wid, QN // L)
        nch = _scalar_at(n_v, wid, QN // L)
        for i in range(ACC // zw):
            pltpu.sync_copy(z_hbm, acc.at[pl.ds(i * zw, zw)])

        def chunk(c, _):
            start = pl.multiple_of(s0 + c * CS, 8)
            pltpu.sync_copy(dst_hbm.at[pl.ds(start, CS)], dst_v)
            pltpu.sync_copy(j_hbm.at[pl.ds(start, CS)], j_v)
            pltpu.sync_copy(rel_hbm.at[pl.ds(start, CS)], rel_v)
            for kk in range(CS // L):
                sl = pl.ds(kk * L, L)
                r16 = rel_v[sl]
                slot_v[sl] = j_v[sl] + plsc.load_gather(dtab_v, [r16])
                d16 = dst_v[sl]
                dloc = d16 - base_node
                inr = (dloc >= 0) & (dloc < NPW)
                lidx_v[sl] = jnp.where(inr, dloc, NPW + (d16 & 63))
            pltpu.async_copy(msg_hbm.at[slot_v], rows_v, sem).wait()

            def grp(g, _2):
                d16 = lidx_v[pl.ds(g * L, L)]
                for j in range(L):
                    row16 = jnp.zeros((L,), jnp.int32) + d16[j]
                    erow16 = jnp.zeros((L,), jnp.int32) + (g * L + j)
                    for cc in range(D // L):
                        col16 = cc * L + lax.iota(jnp.int32, L)
                        vals = plsc.load_gather(rows_v, [erow16, col16])
                        plsc.addupdate_scatter(acc, [row16, col16], vals)
                return _2

            lax.fori_loop(0, CS // L, grp, None)
            return _

        lax.fori_loop(0, nch, chunk, None)

        @pl.when(wid < NW - 1)
        def _full():
            pltpu.sync_copy(
                acc.at[pl.ds(0, NPW)],
                out_hbm.at[pl.ds(base_node, NPW)],
            )

        @pl.when(wid == NW - 1)
        def _tail():
            pltpu.sync_copy(
                acc.at[pl.ds(0, last_rows)],
                out_hbm.at[pl.ds((NW - 1) * NPW, last_rows)],
            )

    return k(msg, dstp, jp, relp, q8, nchv, delta_tab, zrows)


def _tc_finish(acc, norm, x, loop_weight, N, D):
    BR = 400
    grid = (N // BR,)

    def body(acc_ref, norm_ref, x_ref, lw_ref, out_ref):
        out_ref[...] = acc_ref[...] * norm_ref[...] + jnp.dot(
            x_ref[...], lw_ref[...], preferred_element_type=jnp.float32
        )

    return pl.pallas_call(
        body,
        grid=grid,
        in_specs=[
            pl.BlockSpec((BR, D), lambda i: (i, 0)),
            pl.BlockSpec((BR, 1), lambda i: (i, 0)),
            pl.BlockSpec((BR, D), lambda i: (i, 0)),
            pl.BlockSpec((D, D), lambda i: (0, 0)),
        ],
        out_specs=pl.BlockSpec((BR, D), lambda i: (i, 0)),
        out_shape=jax.ShapeDtypeStruct((N, D), jnp.float32),
    )(acc, norm, x, loop_weight)


def kernel(x, edge_index, rel, norm, weight, loop_weight):
    N, D = x.shape
    E = edge_index.shape[1]
    R = weight.shape[0]
    NB = 4
    SI = SO = D // NB
    NT = E // T + R + 1
    Ep = NT * T

    NPW = (-(-N // NW) + 7) // 8 * 8

    src = edge_index[0]
    dst = edge_index[1]
    src_p, rel_p, rel_s, dst_s, delta_tab, tile_rel = _plan(
        rel, src, dst, R, E, Ep)
    dstp, jp, relp, q8, nchv = _plan_dst(dst_s, rel_s, N, E, NPW)
    w4 = weight.reshape(R, NB, SI, SO)
    zrows = jnp.zeros((64, D), jnp.float32)

    xs = _sc_gather(x, src_p, rel_p, delta_tab, E, Ep, D)
    msg = _tc_bmm(xs, w4, tile_rel, Ep, D, NB, SI, SO)
    acc = _sc_scatter(msg, dstp, jp, relp, q8, nchv, delta_tab, zrows,
                      N, E, D, NPW)
    return _tc_finish(acc, norm, x, loop_weight, N, D)


# speedup vs baseline: 16.4627x; 1.0845x over previous
import functools

import jax
import jax.numpy as jnp
from jax import lax
from jax.experimental import pallas as pl
from jax.experimental.pallas import tpu as pltpu
from jax.experimental.pallas import tpu_sc as plsc

NC = 2
NS = 16
NW = NC * NS
L = 16

T = 256
CH = 128
CS = 80


def _plan(rel, src, dst, R, E, Ep):
    key, dst_s = lax.sort(((rel << 14) | src, dst), num_keys=1)
    rel_s = key >> 14
    src_s = key & 0x3FFF
    bounds = jnp.searchsorted(
        rel_s, jnp.arange(R + 1, dtype=jnp.int32), side="left"
    ).astype(jnp.int32)
    off = bounds[:-1]
    counts = bounds[1:] - off
    nt = (counts + (T - 1)) // T
    tile_cum = jnp.cumsum(nt).astype(jnp.int32)
    padded_off = (tile_cum - nt) * T
    delta = padded_off - off
    NT = Ep // T
    tail_delta = jnp.full((1,), Ep - CH - E, dtype=jnp.int32)
    delta_tab = jnp.concatenate([delta, tail_delta, jnp.zeros((7,), jnp.int32)])
    tile_rel = jnp.minimum(
        jnp.searchsorted(tile_cum, jnp.arange(NT, dtype=jnp.int32), side="right"),
        R - 1,
    ).astype(jnp.int32)
    src_p = jnp.concatenate([src_s, jnp.zeros((CH,), jnp.int32)])
    rel_p = jnp.concatenate([rel_s, jnp.full((CH,), R, jnp.int32)])
    return src_p, rel_p, rel_s, dst_s, delta_tab, tile_rel


def _plan_dst(dst_s, rel_s, N, E, NPW):
    iota = jnp.arange(E, dtype=jnp.int32)
    key2, rel_d = lax.sort(
        ((dst_s.astype(jnp.uint32) << 18) | iota.astype(jnp.uint32), rel_s),
        num_keys=1)
    dst_d = (key2 >> 18).astype(jnp.int32)
    j_d = (key2 & 0x3FFFF).astype(jnp.int32)
    q = jnp.searchsorted(
        dst_d, jnp.arange(NW + 1, dtype=jnp.int32) * NPW, side="left"
    ).astype(jnp.int32)
    q8 = (q[:NW] // 16) * 16
    nchv = (q[1:] - q8 + CS - 1) // CS
    QN = 48
    zpad = jnp.zeros((QN - NW,), jnp.int32)
    q8 = jnp.concatenate([q8, zpad])
    nchv = jnp.concatenate([nchv, zpad])
    dstp = jnp.concatenate([dst_d, jnp.full((CS,), N, jnp.int32)])
    jp = jnp.concatenate([j_d, jnp.zeros((CS,), jnp.int32)])
    relp = jnp.concatenate([rel_d, jnp.zeros((CS,), jnp.int32)])
    return dstp, jp, relp, q8, nchv


def _sc_gather(x, src_p, rel_p, delta_tab, E, Ep, D):
    per_w = -(-(-(-E // NW)) // CH) * CH
    nch_full = per_w // CH
    RT = delta_tab.shape[0]
    mesh = plsc.VectorSubcoreMesh(core_axis_name="c", subcore_axis_name="s")

    @functools.partial(
        pl.kernel,
        out_type=jax.ShapeDtypeStruct((Ep, D), jnp.float32),
        mesh=mesh,
        scratch_types=[
            pltpu.VMEM((CH,), jnp.int32),
            pltpu.VMEM((CH,), jnp.int32),
            pltpu.VMEM((1, CH), jnp.int32),
            pltpu.VMEM((RT,), jnp.int32),
            pltpu.VMEM((CH, D), jnp.float32),
            pltpu.SemaphoreType.DMA,
        ],
        compiler_params=pltpu.CompilerParams(needs_layout_passes=False),
    )
    def k(x_hbm, src_hbm, rel_hbm, dtab_hbm, xs_hbm,
          src_v, rel_v, slot_v, dtab_v, rows_v, sem):
        wid = lax.axis_index("s") * NC + lax.axis_index("c")
        base = wid * per_w
        rem = jnp.maximum(E - base, 0)
        nch = jnp.minimum(nch_full, (rem + CH - 1) // CH)
        pltpu.sync_copy(dtab_hbm, dtab_v)

        def chunk(c, _):
            start = pl.multiple_of(base + c * CH, CH)
            pltpu.sync_copy(src_hbm.at[pl.ds(start, CH)], src_v)
            pltpu.sync_copy(rel_hbm.at[pl.ds(start, CH)], rel_v)
            for kk in range(CH // L):
                r16 = rel_v[pl.ds(kk * L, L)]
                d16 = plsc.load_gather(dtab_v, [r16])
                slot_v[0, pl.ds(kk * L, L)] = (
                    d16 + start + kk * L + lax.iota(jnp.int32, L)
                )
            pltpu.async_copy(x_hbm.at[src_v], rows_v, sem).wait()
            pltpu.async_copy(rows_v, xs_hbm.at[slot_v.at[0]], sem).wait()
            return _

        lax.fori_loop(0, nch, chunk, None)

    return k(x, src_p, rel_p, delta_tab)


def _tc_bmm(xs, w4, tile_rel, Ep, D, NB, SI, SO):
    NT = Ep // T

    def body(tr_ref, xs_ref, w_ref, out_ref):
        for b in range(NB):
            out_ref[:, b * SO:(b + 1) * SO] = jnp.dot(
                xs_ref[:, b * SI:(b + 1) * SI],
                w_ref[0, b],
                preferred_element_type=jnp.float32,
            )

    grid_spec = pltpu.PrefetchScalarGridSpec(
        num_scalar_prefetch=1,
        grid=(NT,),
        in_specs=[
            pl.BlockSpec((T, D), lambda t, tr: (t, 0)),
            pl.BlockSpec((1, NB, SI, SO), lambda t, tr: (tr[t], 0, 0, 0)),
        ],
        out_specs=pl.BlockSpec((T, D), lambda t, tr: (t, 0)),
    )
    return pl.pallas_call(
        body,
        grid_spec=grid_spec,
        out_shape=jax.ShapeDtypeStruct((Ep, D), jnp.float32),
    )(tile_rel, xs, w4)


def _scalar_at(vref, i, ngroups):
    total = jnp.zeros((), jnp.int32)
    for g in range(ngroups):
        v = vref[pl.ds(g * L, L)]
        lane = lax.iota(jnp.int32, L) + g * L
        total = total + jnp.sum(jnp.where(lane == i, v, 0))
    return total


def _sc_scatter(msg, dstp, jp, relp, q8, nchv, delta_tab, zrows, N, E, D, NPW):
    RT = delta_tab.shape[0]
    QN = q8.shape[0]
    ACC = NPW + 16
    zw = zrows.shape[0]
    last_rows = N - (NW - 1) * NPW
    mesh = plsc.VectorSubcoreMesh(core_axis_name="c", subcore_axis_name="s")

    @functools.partial(
        pl.kernel,
        out_type=jax.ShapeDtypeStruct((N, D), jnp.float32),
        mesh=mesh,
        scratch_types=[
            pltpu.VMEM((CS,), jnp.int32),
            pltpu.VMEM((CS,), jnp.int32),
            pltpu.VMEM((CS,), jnp.int32),
            pltpu.VMEM((CS,), jnp.int32),
            pltpu.VMEM((CS,), jnp.int32),
            pltpu.VMEM((CS,), jnp.int32),
            pltpu.VMEM((CS,), jnp.int32),
            pltpu.VMEM((RT,), jnp.int32),
            pltpu.VMEM((QN,), jnp.int32),
            pltpu.VMEM((QN,), jnp.int32),
            pltpu.VMEM((CS, D), jnp.float32),
            pltpu.VMEM((CS, D), jnp.float32),
            pltpu.VMEM((ACC, D), jnp.float32),
            pltpu.SemaphoreType.DMA,
            pltpu.SemaphoreType.DMA,
        ],
        compiler_params=pltpu.CompilerParams(needs_layout_passes=False),
    )
    def k(msg_hbm, dst_hbm, j_hbm, rel_hbm, q8_hbm, nch_hbm, dtab_hbm, z_hbm,
          out_hbm, dst_v, j_v, rel_v, slot_a, slot_b, lidx_a, lidx_b,
          dtab_v, q_v, n_v, rows_a, rows_b, acc, sem_a, sem_b):
        wid = lax.axis_index("s") * NC + lax.axis_index("c")
        base_node = wid * NPW
        pltpu.sync_copy(dtab_hbm, dtab_v)
        pltpu.sync_copy(q8_hbm, q_v)
        pltpu.sync_copy(nch_hbm, n_v)
        s0 = _scalar_at(q_v, wid, QN // L)
        nch = _scalar_at(n_v, wid, QN // L)
        for i in range(NPW // zw):
            pltpu.sync_copy(z_hbm, acc.at[pl.ds(i * zw, zw)])
        pltpu.sync_copy(z_hbm.at[pl.ds(0, ACC - NPW)],
                        acc.at[pl.ds(NPW, ACC - NPW)])

        def fire(c, slot_v, lidx_v, rows_v, sem):
            start = pl.multiple_of(s0 + c * CS, 16)
            pltpu.sync_copy(dst_hbm.at[pl.ds(start, CS)], dst_v)
            pltpu.sync_copy(j_hbm.at[pl.ds(start, CS)], j_v)
            pltpu.sync_copy(rel_hbm.at[pl.ds(start, CS)], rel_v)
            for kk in range(CS // L):
                sl = pl.ds(kk * L, L)
                r16 = rel_v[sl]
                slot_v[sl] = j_v[sl] + plsc.load_gather(dtab_v, [r16])
                d16 = dst_v[sl]
                dloc = d16 - base_node
                inr = (dloc >= 0) & (dloc < NPW)
                lidx_v[sl] = jnp.where(inr, dloc, NPW + (d16 & 15))
            return pltpu.async_copy(msg_hbm.at[slot_v], rows_v, sem)

        def drain(lidx_v, rows_v, sem):
            pltpu.make_async_copy(msg_hbm.at[slot_a], rows_v, sem).wait()
            def grp(g, _2):
                d16 = lidx_v[pl.ds(g * L, L)]
                for j in range(L):
                    row16 = jnp.zeros((L,), jnp.int32) + d16[j]
                    erow16 = jnp.zeros((L,), jnp.int32) + (g * L + j)
                    for cc in range(D // L):
                        col16 = cc * L + lax.iota(jnp.int32, L)
                        vals = plsc.load_gather(rows_v, [erow16, col16])
                        plsc.addupdate_scatter(acc, [row16, col16], vals)
                return _2
            lax.fori_loop(0, CS // L, grp, None)

        @pl.when(nch > 0)
        def _prologue():
            fire(0, slot_a, lidx_a, rows_a, sem_a)

        def pair(p, _):
            c0 = 2 * p
            c1 = c0 + 1
            @pl.when(c1 < nch)
            def _fb():
                fire(c1, slot_b, lidx_b, rows_b, sem_b)
            drain(lidx_a, rows_a, sem_a)
            @pl.when(c1 < nch)
            def _db():
                @pl.when(c1 + 1 < nch)
                def _fa():
                    fire(c1 + 1, slot_a, lidx_a, rows_a, sem_a)
                drain(lidx_b, rows_b, sem_b)
            return _

        lax.fori_loop(0, (nch + 1) // 2, pair, None)

        @pl.when(wid < NW - 1)
        def _full():
            pltpu.sync_copy(
                acc.at[pl.ds(0, NPW)],
                out_hbm.at[pl.ds(base_node, NPW)],
            )

        @pl.when(wid == NW - 1)
        def _tail():
            pltpu.sync_copy(
                acc.at[pl.ds(0, last_rows)],
                out_hbm.at[pl.ds((NW - 1) * NPW, last_rows)],
            )

    return k(msg, dstp, jp, relp, q8, nchv, delta_tab, zrows)


def _tc_finish(acc, norm, x, loop_weight, N, D):
    BR = 400
    grid = (N // BR,)

    def body(acc_ref, norm_ref, x_ref, lw_ref, out_ref):
        out_ref[...] = acc_ref[...] * norm_ref[...] + jnp.dot(
            x_ref[...], lw_ref[...], preferred_element_type=jnp.float32
        )

    return pl.pallas_call(
        body,
        grid=grid,
        in_specs=[
            pl.BlockSpec((BR, D), lambda i: (i, 0)),
            pl.BlockSpec((BR, 1), lambda i: (i, 0)),
            pl.BlockSpec((BR, D), lambda i: (i, 0)),
            pl.BlockSpec((D, D), lambda i: (0, 0)),
        ],
        out_specs=pl.BlockSpec((BR, D), lambda i: (i, 0)),
        out_shape=jax.ShapeDtypeStruct((N, D), jnp.float32),
    )(acc, norm, x, loop_weight)


def kernel(x, edge_index, rel, norm, weight, loop_weight):
    N, D = x.shape
    E = edge_index.shape[1]
    R = weight.shape[0]
    NB = 4
    SI = SO = D // NB
    NT = E // T + R + 1
    Ep = NT * T

    NPW = (-(-N // NW) + 7) // 8 * 8

    src = edge_index[0]
    dst = edge_index[1]
    src_p, rel_p, rel_s, dst_s, delta_tab, tile_rel = _plan(
        rel, src, dst, R, E, Ep)
    dstp, jp, relp, q8, nchv = _plan_dst(dst_s, rel_s, N, E, NPW)
    w4 = weight.reshape(R, NB, SI, SO)
    zrows = jnp.zeros((64, D), jnp.float32)

    xs = _sc_gather(x, src_p, rel_p, delta_tab, E, Ep, D)
    msg = _tc_bmm(xs, w4, tile_rel, Ep, D, NB, SI, SO)
    acc = _sc_scatter(msg, dstp, jp, relp, q8, nchv, delta_tab, zrows,
                      N, E, D, NPW)
    return _tc_finish(acc, norm, x, loop_weight, N, D)


# speedup vs baseline: 16.4725x; 1.0006x over previous
import functools

import jax
import jax.numpy as jnp
from jax import lax
from jax.experimental import pallas as pl
from jax.experimental.pallas import tpu as pltpu
from jax.experimental.pallas import tpu_sc as plsc

NC = 2
NS = 16
NW = NC * NS
L = 16

T = 256
CH = 128
CS = 80


def _plan(rel, src, dst, R, E, Ep):
    key, dst_s = lax.sort(((rel << 14) | src, dst), num_keys=1)
    rel_s = key >> 14
    src_s = key & 0x3FFF
    bounds = jnp.searchsorted(
        rel_s, jnp.arange(R + 1, dtype=jnp.int32), side="left"
    ).astype(jnp.int32)
    off = bounds[:-1]
    counts = bounds[1:] - off
    nt = (counts + (T - 1)) // T
    tile_cum = jnp.cumsum(nt).astype(jnp.int32)
    padded_off = (tile_cum - nt) * T
    delta = padded_off - off
    NT = Ep // T
    tail_delta = jnp.full((1,), Ep - CH - E, dtype=jnp.int32)
    delta_tab = jnp.concatenate([delta, tail_delta, jnp.zeros((7,), jnp.int32)])
    tile_rel = jnp.minimum(
        jnp.searchsorted(tile_cum, jnp.arange(NT, dtype=jnp.int32), side="right"),
        R - 1,
    ).astype(jnp.int32)
    src_p = jnp.concatenate([src_s, jnp.zeros((CH,), jnp.int32)])
    rel_p = jnp.concatenate([rel_s, jnp.full((CH,), R, jnp.int32)])
    return src_p, rel_p, rel_s, dst_s, delta_tab, tile_rel


def _plan_dst(dst_s, rel_s, N, E, NPW):
    iota = jnp.arange(E, dtype=jnp.int32)
    key2, rel_d = lax.sort(
        ((dst_s.astype(jnp.uint32) << 18) | iota.astype(jnp.uint32), rel_s),
        num_keys=1)
    dst_d = (key2 >> 18).astype(jnp.int32)
    j_d = (key2 & 0x3FFFF).astype(jnp.int32)
    q = jnp.searchsorted(
        dst_d, jnp.arange(NW + 1, dtype=jnp.int32) * NPW, side="left"
    ).astype(jnp.int32)
    q8 = (q[:NW] // 16) * 16
    nchv = (q[1:] - q8 + CS - 1) // CS
    QN = 48
    zpad = jnp.zeros((QN - NW,), jnp.int32)
    q8 = jnp.concatenate([q8, zpad])
    nchv = jnp.concatenate([nchv, zpad])
    dstp = jnp.concatenate([dst_d, jnp.full((CS,), N, jnp.int32)])
    jp = jnp.concatenate([j_d, jnp.zeros((CS,), jnp.int32)])
    relp = jnp.concatenate([rel_d, jnp.zeros((CS,), jnp.int32)])
    return dstp, jp, relp, q8, nchv


def _sc_gather(x, src_p, rel_p, delta_tab, E, Ep, D):
    per_w = -(-(-(-E // NW)) // CH) * CH
    nch_full = per_w // CH
    RT = delta_tab.shape[0]
    mesh = plsc.VectorSubcoreMesh(core_axis_name="c", subcore_axis_name="s")

    @functools.partial(
        pl.kernel,
        out_type=jax.ShapeDtypeStruct((Ep, D), jnp.float32),
        mesh=mesh,
        scratch_types=[
            pltpu.VMEM((CH,), jnp.int32),
            pltpu.VMEM((CH,), jnp.int32),
            pltpu.VMEM((CH,), jnp.int32),
            pltpu.VMEM((1, CH), jnp.int32),
            pltpu.VMEM((1, CH), jnp.int32),
            pltpu.VMEM((RT,), jnp.int32),
            pltpu.VMEM((CH, D), jnp.float32),
            pltpu.VMEM((CH, D), jnp.float32),
            pltpu.SemaphoreType.DMA,
            pltpu.SemaphoreType.DMA,
            pltpu.SemaphoreType.DMA,
            pltpu.SemaphoreType.DMA,
        ],
        compiler_params=pltpu.CompilerParams(needs_layout_passes=False),
    )
    def k(x_hbm, src_hbm, rel_hbm, dtab_hbm, xs_hbm,
          src_a, src_b, rel_v, slot_a, slot_b, dtab_v,
          rows_a, rows_b, gsem_a, gsem_b, ssem_a, ssem_b):
        wid = lax.axis_index("s") * NC + lax.axis_index("c")
        base = wid * per_w
        rem = jnp.maximum(E - base, 0)
        nch = jnp.minimum(nch_full, (rem + CH - 1) // CH)
        pltpu.sync_copy(dtab_hbm, dtab_v)

        def fire(c, src_v, slot_v, rows_v, gsem):
            start = pl.multiple_of(base + c * CH, CH)
            pltpu.sync_copy(src_hbm.at[pl.ds(start, CH)], src_v)
            pltpu.sync_copy(rel_hbm.at[pl.ds(start, CH)], rel_v)
            for kk in range(CH // L):
                r16 = rel_v[pl.ds(kk * L, L)]
                d16 = plsc.load_gather(dtab_v, [r16])
                slot_v[0, pl.ds(kk * L, L)] = (
                    d16 + start + kk * L + lax.iota(jnp.int32, L)
                )
            pltpu.async_copy(x_hbm.at[src_v], rows_v, gsem)

        def scat(src_v, slot_v, rows_v, gsem, ssem):
            pltpu.make_async_copy(x_hbm.at[src_v], rows_v, gsem).wait()
            pltpu.async_copy(rows_v, xs_hbm.at[slot_v.at[0]], ssem)

        def swait(slot_v, rows_v, ssem):
            pltpu.make_async_copy(rows_v, xs_hbm.at[slot_v.at[0]], ssem).wait()

        @pl.when(nch > 0)
        def _pro():
            fire(0, src_a, slot_a, rows_a, gsem_a)

        def pair(p, _):
            c0 = 2 * p
            c1 = c0 + 1
            @pl.when(c1 < nch)
            def _fb():
                fire(c1, src_b, slot_b, rows_b, gsem_b)
            scat(src_a, slot_a, rows_a, gsem_a, ssem_a)
            @pl.when(c1 < nch)
            def _db():
                scat(src_b, slot_b, rows_b, gsem_b, ssem_b)
                swait(slot_a, rows_a, ssem_a)
                @pl.when(c1 + 1 < nch)
                def _fa():
                    fire(c1 + 1, src_a, slot_a, rows_a, gsem_a)
                swait(slot_b, rows_b, ssem_b)
            @pl.when(c1 >= nch)
            def _dlast():
                swait(slot_a, rows_a, ssem_a)
            return _

        lax.fori_loop(0, (nch + 1) // 2, pair, None)

    return k(x, src_p, rel_p, delta_tab)


def _tc_bmm(xs, w4, tile_rel, Ep, D, NB, SI, SO):
    NT = Ep // T

    def body(tr_ref, xs_ref, w_ref, out_ref):
        for b in range(NB):
            out_ref[:, b * SO:(b + 1) * SO] = jnp.dot(
                xs_ref[:, b * SI:(b + 1) * SI],
                w_ref[0, b],
                preferred_element_type=jnp.float32,
            )

    grid_spec = pltpu.PrefetchScalarGridSpec(
        num_scalar_prefetch=1,
        grid=(NT,),
        in_specs=[
            pl.BlockSpec((T, D), lambda t, tr: (t, 0)),
            pl.BlockSpec((1, NB, SI, SO), lambda t, tr: (tr[t], 0, 0, 0)),
        ],
        out_specs=pl.BlockSpec((T, D), lambda t, tr: (t, 0)),
    )
    return pl.pallas_call(
        body,
        grid_spec=grid_spec,
        out_shape=jax.ShapeDtypeStruct((Ep, D), jnp.float32),
    )(tile_rel, xs, w4)


def _scalar_at(vref, i, ngroups):
    total = jnp.zeros((), jnp.int32)
    for g in range(ngroups):
        v = vref[pl.ds(g * L, L)]
        lane = lax.iota(jnp.int32, L) + g * L
        total = total + jnp.sum(jnp.where(lane == i, v, 0))
    return total


def _sc_scatter(msg, dstp, jp, relp, q8, nchv, delta_tab, zrows, N, E, D, NPW):
    RT = delta_tab.shape[0]
    QN = q8.shape[0]
    ACC = NPW + 16
    zw = zrows.shape[0]
    last_rows = N - (NW - 1) * NPW
    mesh = plsc.VectorSubcoreMesh(core_axis_name="c", subcore_axis_name="s")

    @functools.partial(
        pl.kernel,
        out_type=jax.ShapeDtypeStruct((N, D), jnp.float32),
        mesh=mesh,
        scratch_types=[
            pltpu.VMEM((CS,), jnp.int32),
            pltpu.VMEM((CS,), jnp.int32),
            pltpu.VMEM((CS,), jnp.int32),
            pltpu.VMEM((CS,), jnp.int32),
            pltpu.VMEM((CS,), jnp.int32),
            pltpu.VMEM((CS,), jnp.int32),
            pltpu.VMEM((CS,), jnp.int32),
            pltpu.VMEM((RT,), jnp.int32),
            pltpu.VMEM((QN,), jnp.int32),
            pltpu.VMEM((QN,), jnp.int32),
            pltpu.VMEM((CS, D), jnp.float32),
            pltpu.VMEM((CS, D), jnp.float32),
            pltpu.VMEM((ACC, D), jnp.float32),
            pltpu.SemaphoreType.DMA,
            pltpu.SemaphoreType.DMA,
        ],
        compiler_params=pltpu.CompilerParams(needs_layout_passes=False),
    )
    def k(msg_hbm, dst_hbm, j_hbm, rel_hbm, q8_hbm, nch_hbm, dtab_hbm, z_hbm,
          out_hbm, dst_v, j_v, rel_v, slot_a, slot_b, lidx_a, lidx_b,
          dtab_v, q_v, n_v, rows_a, rows_b, acc, sem_a, sem_b):
        wid = lax.axis_index("s") * NC + lax.axis_index("c")
        base_node = wid * NPW
        pltpu.sync_copy(dtab_hbm, dtab_v)
        pltpu.sync_copy(q8_hbm, q_v)
        pltpu.sync_copy(nch_hbm, n_v)
        s0 = _scalar_at(q_v, wid, QN // L)
        nch = _scalar_at(n_v, wid, QN // L)
        for i in range(NPW // zw):
            pltpu.sync_copy(z_hbm, acc.at[pl.ds(i * zw, zw)])
        pltpu.sync_copy(z_hbm.at[pl.ds(0, ACC - NPW)],
                        acc.at[pl.ds(NPW, ACC - NPW)])

        def fire(c, slot_v, lidx_v, rows_v, sem):
            start = pl.multiple_of(s0 + c * CS, 16)
            pltpu.sync_copy(dst_hbm.at[pl.ds(start, CS)], dst_v)
            pltpu.sync_copy(j_hbm.at[pl.ds(start, CS)], j_v)
            pltpu.sync_copy(rel_hbm.at[pl.ds(start, CS)], rel_v)
            for kk in range(CS // L):
                sl = pl.ds(kk * L, L)
                r16 = rel_v[sl]
                slot_v[sl] = j_v[sl] + plsc.load_gather(dtab_v, [r16])
                d16 = dst_v[sl]
                dloc = d16 - base_node
                inr = (dloc >= 0) & (dloc < NPW)
                lidx_v[sl] = jnp.where(inr, dloc, NPW + (d16 & 15))
            return pltpu.async_copy(msg_hbm.at[slot_v], rows_v, sem)

        def drain(lidx_v, rows_v, sem):
            pltpu.make_async_copy(msg_hbm.at[slot_a], rows_v, sem).wait()
            def grp(g, _2):
                d16 = lidx_v[pl.ds(g * L, L)]
                for j in range(L):
                    row16 = jnp.zeros((L,), jnp.int32) + d16[j]
                    erow16 = jnp.zeros((L,), jnp.int32) + (g * L + j)
                    for cc in range(D // L):
                        col16 = cc * L + lax.iota(jnp.int32, L)
                        vals = plsc.load_gather(rows_v, [erow16, col16])
                        plsc.addupdate_scatter(acc, [row16, col16], vals)
                return _2
            lax.fori_loop(0, CS // L, grp, None)

        @pl.when(nch > 0)
        def _prologue():
            fire(0, slot_a, lidx_a, rows_a, sem_a)

        def pair(p, _):
            c0 = 2 * p
            c1 = c0 + 1
            @pl.when(c1 < nch)
            def _fb():
                fire(c1, slot_b, lidx_b, rows_b, sem_b)
            drain(lidx_a, rows_a, sem_a)
            @pl.when(c1 < nch)
            def _db():
                @pl.when(c1 + 1 < nch)
                def _fa():
                    fire(c1 + 1, slot_a, lidx_a, rows_a, sem_a)
                drain(lidx_b, rows_b, sem_b)
            return _

        lax.fori_loop(0, (nch + 1) // 2, pair, None)

        @pl.when(wid < NW - 1)
        def _full():
            pltpu.sync_copy(
                acc.at[pl.ds(0, NPW)],
                out_hbm.at[pl.ds(base_node, NPW)],
            )

        @pl.when(wid == NW - 1)
        def _tail():
            pltpu.sync_copy(
                acc.at[pl.ds(0, last_rows)],
                out_hbm.at[pl.ds((NW - 1) * NPW, last_rows)],
            )

    return k(msg, dstp, jp, relp, q8, nchv, delta_tab, zrows)


def _tc_finish(acc, norm, x, loop_weight, N, D):
    BR = 400
    grid = (N // BR,)

    def body(acc_ref, norm_ref, x_ref, lw_ref, out_ref):
        out_ref[...] = acc_ref[...] * norm_ref[...] + jnp.dot(
            x_ref[...], lw_ref[...], preferred_element_type=jnp.float32
        )

    return pl.pallas_call(
        body,
        grid=grid,
        in_specs=[
            pl.BlockSpec((BR, D), lambda i: (i, 0)),
            pl.BlockSpec((BR, 1), lambda i: (i, 0)),
            pl.BlockSpec((BR, D), lambda i: (i, 0)),
            pl.BlockSpec((D, D), lambda i: (0, 0)),
        ],
        out_specs=pl.BlockSpec((BR, D), lambda i: (i, 0)),
        out_shape=jax.ShapeDtypeStruct((N, D), jnp.float32),
    )(acc, norm, x, loop_weight)


def kernel(x, edge_index, rel, norm, weight, loop_weight):
    N, D = x.shape
    E = edge_index.shape[1]
    R = weight.shape[0]
    NB = 4
    SI = SO = D // NB
    NT = E // T + R + 1
    Ep = NT * T

    NPW = (-(-N // NW) + 7) // 8 * 8

    src = edge_index[0]
    dst = edge_index[1]
    src_p, rel_p, rel_s, dst_s, delta_tab, tile_rel = _plan(
        rel, src, dst, R, E, Ep)
    dstp, jp, relp, q8, nchv = _plan_dst(dst_s, rel_s, N, E, NPW)
    w4 = weight.reshape(R, NB, SI, SO)
    zrows = jnp.zeros((64, D), jnp.float32)

    xs = _sc_gather(x, src_p, rel_p, delta_tab, E, Ep, D)
    msg = _tc_bmm(xs, w4, tile_rel, Ep, D, NB, SI, SO)
    acc = _sc_scatter(msg, dstp, jp, relp, q8, nchv, delta_tab, zrows,
                      N, E, D, NPW)
    return _tc_finish(acc, norm, x, loop_weight, N, D)
